# baseline (speedup 1.0000x reference)
import functools

import jax
import jax.numpy as jnp
from jax import lax
from jax.experimental import pallas as pl
from jax.experimental.pallas import tpu as pltpu
from jax.experimental.pallas import tpu_sc as plsc

T, D, E, FF = 2048, 2048, 8, 1024
K = 2
A = T * K
BT = 256
BR = 128
MT = A // BR + E - 1
NPAD = MT * BR
NW = 32
CH = 32
CT = 16


def _router_body(x_ref, rw_ref, mi_ref, mf_ref, cnts_ref, cnt_ref):
    i = pl.program_id(0)

    @pl.when(i == 0)
    def _():
        cnt_ref[...] = jnp.zeros_like(cnt_ref)

    x = x_ref[...]
    rw = rw_ref[...]
    logits = jax.lax.dot_general(
        x, rw, (((1,), (1,)), ((), ())), preferred_element_type=jnp.float32)
    m = jnp.max(logits, axis=-1, keepdims=True)
    p = jnp.exp(logits - m)
    p = p / jnp.sum(p, axis=-1, keepdims=True)
    ii = lax.broadcasted_iota(jnp.int32, p.shape, 1)
    m1 = jnp.max(p, axis=-1, keepdims=True)
    i1 = jnp.min(jnp.where(p >= m1, ii, E), axis=-1, keepdims=True)
    p2 = jnp.where(ii == i1, -jnp.inf, p)
    m2 = jnp.max(p2, axis=-1, keepdims=True)
    i2 = jnp.min(jnp.where(p2 >= m2, ii, E), axis=-1, keepdims=True)
    s = m1 + m2
    w1 = m1 / s
    w2 = m2 / s

    ae = jnp.concatenate([i1, i2], axis=0)
    col = lax.broadcasted_iota(jnp.int32, (2 * BT, 128), 1)
    oh = (col == ae).astype(jnp.float32)
    r_io = lax.broadcasted_iota(jnp.int32, (2 * BT, 2 * BT), 0)
    c_io = lax.broadcasted_iota(jnp.int32, (2 * BT, 2 * BT), 1)
    tril = (c_io < r_io).astype(jnp.float32)
    local = jax.lax.dot_general(
        tril, oh, (((1,), (0,)), ((), ())), preferred_element_type=jnp.float32)
    base = cnt_ref[...]
    rk = local + base[0:1, :]
    rank = jnp.sum(jnp.where(col == ae, rk, 0.0), axis=1, keepdims=True)
    r1 = rank[:BT].astype(jnp.int32)
    r2 = rank[BT:].astype(jnp.int32)

    mcol = lax.broadcasted_iota(jnp.int32, (BT, 128), 1)
    mi_ref[...] = (jnp.where(mcol == 0, i1, 0) + jnp.where(mcol == 1, i2, 0)
                   + jnp.where(mcol == 2, r1, 0) + jnp.where(mcol == 3, r2, 0))
    mf_ref[...] = jnp.where(mcol == 0, w1, 0.0) + jnp.where(mcol == 1, w2, 0.0)

    newcnt = base + jnp.broadcast_to(
        jnp.sum(oh, axis=0, keepdims=True), base.shape)
    cnt_ref[...] = newcnt
    cnts_ref[...] = newcnt.astype(jnp.int32)


def _router(x, rw):
    return pl.pallas_call(
        _router_body,
        grid=(T // BT,),
        in_specs=[
            pl.BlockSpec((BT, D), lambda i: (i, 0)),
            pl.BlockSpec((E, D), lambda i: (0, 0)),
        ],
        out_specs=[
            pl.BlockSpec((BT, 128), lambda i: (i, 0)),
            pl.BlockSpec((BT, 128), lambda i: (i, 0)),
            pl.BlockSpec((8, 128), lambda i: (0, 0)),
        ],
        out_shape=[
            jax.ShapeDtypeStruct((T, 128), jnp.int32),
            jax.ShapeDtypeStruct((T, 128), jnp.float32),
            jax.ShapeDtypeStruct((8, 128), jnp.int32),
        ],
        scratch_shapes=[pltpu.VMEM((8, 128), jnp.float32)],
    )(x, rw)


def _dispatch(xb, dest, wsplat):
    mesh = plsc.VectorSubcoreMesh(core_axis_name="c", subcore_axis_name="s")

    @functools.partial(
        pl.kernel,
        mesh=mesh,
        out_type=[
            jax.ShapeDtypeStruct((NPAD, D // 2), jnp.int32),
            jax.ShapeDtypeStruct((NPAD, 128), jnp.float32),
        ],
        scratch_types=[
            pltpu.VMEM((CH, D // 2), jnp.int32),
            pltpu.VMEM((CH, 128), jnp.float32),
            pltpu.VMEM((CH,), jnp.int32),
            pltpu.SemaphoreType.DMA,
            pltpu.SemaphoreType.DMA,
        ],
    )
    def k(xb_hbm, dest_hbm, ws_hbm, xs_hbm, wp_hbm, rows_v, ws_v, idx_v,
          sem0, sem1):
        wid = lax.axis_index("s") * 2 + lax.axis_index("c")
        jbase = wid * (A // NW)
        for c in range(A // NW // CH):
            off = jbase + c * CH
            pltpu.sync_copy(dest_hbm.at[pl.ds(off, CH)], idx_v)
            pltpu.sync_copy(xb_hbm.at[pl.ds(off % T, CH)], rows_v)
            pltpu.sync_copy(ws_hbm.at[pl.ds(off, CH)], ws_v)
            cp0 = pltpu.async_copy(rows_v, xs_hbm.at[idx_v], sem0)
            cp1 = pltpu.async_copy(ws_v, wp_hbm.at[idx_v], sem1)
            cp0.wait()
            cp1.wait()

    return k(xb, dest, wsplat)


def _gemm_body(te_ref, x_ref, w13_ref, w2_ref, wp_ref, y_ref):
    x = x_ref[...]
    wg = w13_ref[0, :FF, :]
    wu = w13_ref[0, FF:, :]
    g = jax.lax.dot_general(
        x, wg, (((1,), (1,)), ((), ())), preferred_element_type=jnp.float32)
    u = jax.lax.dot_general(
        x, wu, (((1,), (1,)), ((), ())), preferred_element_type=jnp.float32)
    a = ((g * jax.nn.sigmoid(g)) * u).astype(jnp.bfloat16)
    y = jax.lax.dot_general(
        a, w2_ref[0], (((1,), (1,)), ((), ())),
        preferred_element_type=jnp.float32)
    wcol = lax.broadcasted_iota(jnp.int32, (BR, 128), 1)
    w = jnp.sum(jnp.where(wcol == 0, wp_ref[...], 0.0), axis=1, keepdims=True)
    y_ref[...] = y * w


def _gemm(te, xs, w13b, w2b, wpad):
    grid_spec = pltpu.PrefetchScalarGridSpec(
        num_scalar_prefetch=1,
        grid=(MT,),
        in_specs=[
            pl.BlockSpec((BR, D), lambda i, te: (i, 0)),
            pl.BlockSpec((1, 2 * FF, D), lambda i, te: (te[i], 0, 0)),
            pl.BlockSpec((1, D, FF), lambda i, te: (te[i], 0, 0)),
            pl.BlockSpec((BR, 128), lambda i, te: (i, 0)),
        ],
        out_specs=pl.BlockSpec((BR, D), lambda i, te: (i, 0)),
    )
    return pl.pallas_call(
        _gemm_body,
        grid_spec=grid_spec,
        out_shape=jax.ShapeDtypeStruct((NPAD, D), jnp.float32),
    )(te, xs, w13b, w2b, wpad)


def _combine(ys, dest):
    mesh = plsc.VectorSubcoreMesh(core_axis_name="c", subcore_axis_name="s")

    @functools.partial(
        pl.kernel,
        mesh=mesh,
        out_type=jax.ShapeDtypeStruct((T, D), jnp.float32),
        scratch_types=[
            pltpu.VMEM((CT,), jnp.int32),
            pltpu.VMEM((CT,), jnp.int32),
            pltpu.VMEM((CT, D), jnp.float32),
            pltpu.VMEM((CT, D), jnp.float32),
            pltpu.VMEM((CT, D), jnp.float32),
            pltpu.SemaphoreType.DMA,
            pltpu.SemaphoreType.DMA,
        ],
    )
    def k(ys_hbm, dest_hbm, out_hbm, idx0_v, idx1_v, r0_v, r1_v, o_v,
          sem0, sem1):
        wid = lax.axis_index("s") * 2 + lax.axis_index("c")
        tbase = wid * (T // NW)
        for c in range(T // NW // CT):
            tb = tbase + c * CT
            pltpu.sync_copy(dest_hbm.at[pl.ds(tb, CT)], idx0_v)
            pltpu.sync_copy(dest_hbm.at[pl.ds(T + tb, CT)], idx1_v)
            cp0 = pltpu.async_copy(ys_hbm.at[idx0_v], r0_v, sem0)
            cp1 = pltpu.async_copy(ys_hbm.at[idx1_v], r1_v, sem1)
            cp0.wait()
            cp1.wait()
            for r in range(CT):
                def body(cc, _):
                    sl = pl.ds(cc * 16, 16)
                    o_v[r, sl] = r0_v[r, sl] + r1_v[r, sl]
                    return 0
                lax.fori_loop(0, D // 16, body, 0)
            pltpu.sync_copy(o_v, out_hbm.at[pl.ds(tb, CT)])

    return k(ys, dest)


def kernel(hidden_states, router_weight, w13, w2):
    xb = jax.lax.bitcast_convert_type(
        hidden_states.astype(jnp.bfloat16).reshape(T, D // 2, 2), jnp.int32)
    w13b = w13.astype(jnp.bfloat16)
    w2b = w2.astype(jnp.bfloat16)

    mi, mf, cnts = _router(hidden_states, router_weight)
    counts = cnts[0, :E]
    nt = (counts + BR - 1) // BR
    cum = jnp.cumsum(nt)
    po = (cum - nt) * BR
    ae = jnp.concatenate([mi[:, 0], mi[:, 1]])
    rank = jnp.concatenate([mi[:, 2], mi[:, 3]])
    dest = jnp.clip(po[ae] + rank, 0, NPAD - 1).astype(jnp.int32)
    te = jnp.clip(jnp.searchsorted(cum, jnp.arange(MT), side="right"),
                  0, E - 1).astype(jnp.int32)
    wsplat = jnp.broadcast_to(
        jnp.concatenate([mf[:, 0], mf[:, 1]])[:, None], (A, 128))

    xs32, wpad = _dispatch(xb, dest, wsplat)
    xs = jax.lax.bitcast_convert_type(xs32, jnp.bfloat16).reshape(NPAD, D)
    ys = _gemm(te, xs, w13b, w2b, wpad)
    return _combine(ys, dest)


# speedup vs baseline: 1.7046x; 1.7046x over previous
import functools

import jax
import jax.numpy as jnp
from jax import lax
from jax.experimental import pallas as pl
from jax.experimental.pallas import tpu as pltpu
from jax.experimental.pallas import tpu_sc as plsc

T, D, E, FF = 2048, 2048, 8, 1024
K = 2
A = T * K
BT = 256
BR = 128
MT = A // BR + E - 1
NPAD = MT * BR
NW = 32
CH = 32
CT = 16


def _router_body(x_ref, rw_ref, mi_ref, mf_ref, cnts_ref, cnt_ref):
    i = pl.program_id(0)

    @pl.when(i == 0)
    def _():
        cnt_ref[...] = jnp.zeros_like(cnt_ref)

    x = x_ref[...]
    rw = rw_ref[...]
    logits = jax.lax.dot_general(
        x, rw, (((1,), (1,)), ((), ())), preferred_element_type=jnp.float32)
    m = jnp.max(logits, axis=-1, keepdims=True)
    p = jnp.exp(logits - m)
    p = p / jnp.sum(p, axis=-1, keepdims=True)
    ii = lax.broadcasted_iota(jnp.int32, p.shape, 1)
    m1 = jnp.max(p, axis=-1, keepdims=True)
    i1 = jnp.min(jnp.where(p >= m1, ii, E), axis=-1, keepdims=True)
    p2 = jnp.where(ii == i1, -jnp.inf, p)
    m2 = jnp.max(p2, axis=-1, keepdims=True)
    i2 = jnp.min(jnp.where(p2 >= m2, ii, E), axis=-1, keepdims=True)
    s = m1 + m2
    w1 = m1 / s
    w2 = m2 / s

    ae = jnp.concatenate([i1, i2], axis=0)
    col = lax.broadcasted_iota(jnp.int32, (2 * BT, 128), 1)
    oh = (col == ae).astype(jnp.float32)
    r_io = lax.broadcasted_iota(jnp.int32, (2 * BT, 2 * BT), 0)
    c_io = lax.broadcasted_iota(jnp.int32, (2 * BT, 2 * BT), 1)
    tril = (c_io < r_io).astype(jnp.float32)
    local = jax.lax.dot_general(
        tril, oh, (((1,), (0,)), ((), ())), preferred_element_type=jnp.float32)
    base = cnt_ref[...]
    rk = local + base[0:1, :]
    rank = jnp.sum(jnp.where(col == ae, rk, 0.0), axis=1, keepdims=True)
    r1 = rank[:BT].astype(jnp.int32)
    r2 = rank[BT:].astype(jnp.int32)

    mcol = lax.broadcasted_iota(jnp.int32, (BT, 128), 1)
    mi_ref[...] = (jnp.where(mcol == 0, i1, 0) + jnp.where(mcol == 1, i2, 0)
                   + jnp.where(mcol == 2, r1, 0) + jnp.where(mcol == 3, r2, 0))
    mf_ref[...] = jnp.where(mcol == 0, w1, 0.0) + jnp.where(mcol == 1, w2, 0.0)

    newcnt = base + jnp.broadcast_to(
        jnp.sum(oh, axis=0, keepdims=True), base.shape)
    cnt_ref[...] = newcnt
    cnts_ref[...] = newcnt.astype(jnp.int32)


def _router(x, rw):
    return pl.pallas_call(
        _router_body,
        grid=(T // BT,),
        in_specs=[
            pl.BlockSpec((BT, D), lambda i: (i, 0)),
            pl.BlockSpec((E, D), lambda i: (0, 0)),
        ],
        out_specs=[
            pl.BlockSpec((BT, 128), lambda i: (i, 0)),
            pl.BlockSpec((BT, 128), lambda i: (i, 0)),
            pl.BlockSpec((8, 128), lambda i: (0, 0)),
        ],
        out_shape=[
            jax.ShapeDtypeStruct((T, 128), jnp.int32),
            jax.ShapeDtypeStruct((T, 128), jnp.float32),
            jax.ShapeDtypeStruct((8, 128), jnp.int32),
        ],
        scratch_shapes=[pltpu.VMEM((8, 128), jnp.float32)],
    )(x, rw)


def _dispatch(xb, dest, wsplat):
    mesh = plsc.VectorSubcoreMesh(core_axis_name="c", subcore_axis_name="s")

    @functools.partial(
        pl.kernel,
        mesh=mesh,
        out_type=[
            jax.ShapeDtypeStruct((NPAD, D), jnp.float32),
            jax.ShapeDtypeStruct((NPAD, 128), jnp.float32),
        ],
        scratch_types=[
            pltpu.VMEM((CH, D), jnp.float32),
            pltpu.VMEM((CH, 128), jnp.float32),
            pltpu.VMEM((CH,), jnp.int32),
            pltpu.SemaphoreType.DMA,
            pltpu.SemaphoreType.DMA,
        ],
    )
    def k(xb_hbm, dest_hbm, ws_hbm, xs_hbm, wp_hbm, rows_v, ws_v, idx_v,
          sem0, sem1):
        wid = lax.axis_index("s") * 2 + lax.axis_index("c")
        jbase = wid * (A // NW)
        for c in range(A // NW // CH):
            off = jbase + c * CH
            pltpu.sync_copy(dest_hbm.at[pl.ds(off, CH)], idx_v)
            pltpu.sync_copy(xb_hbm.at[pl.ds(off % T, CH)], rows_v)
            pltpu.sync_copy(ws_hbm.at[pl.ds(off, CH)], ws_v)
            cp0 = pltpu.async_copy(rows_v, xs_hbm.at[idx_v], sem0)
            cp1 = pltpu.async_copy(ws_v, wp_hbm.at[idx_v], sem1)
            cp0.wait()
            cp1.wait()

    return k(xb, dest, wsplat)


def _gemm_body(te_ref, x_ref, w13_ref, w2_ref, wp_ref, y_ref):
    x = x_ref[...].astype(jnp.bfloat16)
    wg = w13_ref[0, :FF, :]
    wu = w13_ref[0, FF:, :]
    g = jax.lax.dot_general(
        x, wg, (((1,), (1,)), ((), ())), preferred_element_type=jnp.float32)
    u = jax.lax.dot_general(
        x, wu, (((1,), (1,)), ((), ())), preferred_element_type=jnp.float32)
    a = ((g * jax.nn.sigmoid(g)) * u).astype(jnp.bfloat16)
    y = jax.lax.dot_general(
        a, w2_ref[0], (((1,), (1,)), ((), ())),
        preferred_element_type=jnp.float32)
    wcol = lax.broadcasted_iota(jnp.int32, (BR, 128), 1)
    w = jnp.sum(jnp.where(wcol == 0, wp_ref[...], 0.0), axis=1, keepdims=True)
    y_ref[...] = y * w


def _gemm(te, xs, w13b, w2b, wpad):
    grid_spec = pltpu.PrefetchScalarGridSpec(
        num_scalar_prefetch=1,
        grid=(MT,),
        in_specs=[
            pl.BlockSpec((BR, D), lambda i, te: (i, 0)),
            pl.BlockSpec((1, 2 * FF, D), lambda i, te: (te[i], 0, 0)),
            pl.BlockSpec((1, D, FF), lambda i, te: (te[i], 0, 0)),
            pl.BlockSpec((BR, 128), lambda i, te: (i, 0)),
        ],
        out_specs=pl.BlockSpec((BR, D), lambda i, te: (i, 0)),
    )
    return pl.pallas_call(
        _gemm_body,
        grid_spec=grid_spec,
        out_shape=jax.ShapeDtypeStruct((NPAD, D), jnp.float32),
    )(te, xs, w13b, w2b, wpad)


def _combine(ys, dest):
    mesh = plsc.VectorSubcoreMesh(core_axis_name="c", subcore_axis_name="s")

    @functools.partial(
        pl.kernel,
        mesh=mesh,
        out_type=jax.ShapeDtypeStruct((T, D), jnp.float32),
        scratch_types=[
            pltpu.VMEM((CT,), jnp.int32),
            pltpu.VMEM((CT,), jnp.int32),
            pltpu.VMEM((CT, D), jnp.float32),
            pltpu.VMEM((CT, D), jnp.float32),
            pltpu.VMEM((CT, D), jnp.float32),
            pltpu.SemaphoreType.DMA,
            pltpu.SemaphoreType.DMA,
        ],
    )
    def k(ys_hbm, dest_hbm, out_hbm, idx0_v, idx1_v, r0_v, r1_v, o_v,
          sem0, sem1):
        wid = lax.axis_index("s") * 2 + lax.axis_index("c")
        tbase = wid * (T // NW)
        for c in range(T // NW // CT):
            tb = tbase + c * CT
            pltpu.sync_copy(dest_hbm.at[pl.ds(tb, CT)], idx0_v)
            pltpu.sync_copy(dest_hbm.at[pl.ds(T + tb, CT)], idx1_v)
            cp0 = pltpu.async_copy(ys_hbm.at[idx0_v], r0_v, sem0)
            cp1 = pltpu.async_copy(ys_hbm.at[idx1_v], r1_v, sem1)
            cp0.wait()
            cp1.wait()
            for r in range(CT):
                def body(cc, _):
                    sl = pl.ds(cc * 16, 16)
                    o_v[r, sl] = r0_v[r, sl] + r1_v[r, sl]
                    return 0
                lax.fori_loop(0, D // 16, body, 0)
            pltpu.sync_copy(o_v, out_hbm.at[pl.ds(tb, CT)])

    return k(ys, dest)


def kernel(hidden_states, router_weight, w13, w2):
    w13b = w13.astype(jnp.bfloat16)
    w2b = w2.astype(jnp.bfloat16)

    mi, mf, cnts = _router(hidden_states, router_weight)
    counts = cnts[0, :E]
    nt = (counts + BR - 1) // BR
    cum = jnp.cumsum(nt)
    po = (cum - nt) * BR
    ae = jnp.concatenate([mi[:, 0], mi[:, 1]])
    rank = jnp.concatenate([mi[:, 2], mi[:, 3]])
    dest = jnp.clip(po[ae] + rank, 0, NPAD - 1).astype(jnp.int32)
    te = jnp.clip(jnp.searchsorted(cum, jnp.arange(MT), side="right"),
                  0, E - 1).astype(jnp.int32)
    wsplat = jnp.broadcast_to(
        jnp.concatenate([mf[:, 0], mf[:, 1]])[:, None], (A, 128))

    xs, wpad = _dispatch(hidden_states, dest, wsplat)
    ys = _gemm(te, xs, w13b, w2b, wpad)
    return _combine(ys, dest)


# speedup vs baseline: 2.5274x; 1.4826x over previous
import functools

import jax
import jax.numpy as jnp
from jax import lax
from jax.experimental import pallas as pl
from jax.experimental.pallas import tpu as pltpu
from jax.experimental.pallas import tpu_sc as plsc

T, D, E, FF = 2048, 2048, 8, 1024
K = 2
A = T * K
BT = 256
BR = 256
MT = A // BR + E - 1
NPAD = MT * BR
NW = 32
CH = 32
CT = 16


def _router_body(x_ref, rw_ref, mi_ref, mf_ref, cnts_ref, cnt_ref):
    i = pl.program_id(0)

    @pl.when(i == 0)
    def _():
        cnt_ref[...] = jnp.zeros_like(cnt_ref)

    x = x_ref[...]
    rw = rw_ref[...]
    logits = jax.lax.dot_general(
        x, rw, (((1,), (1,)), ((), ())), preferred_element_type=jnp.float32)
    m = jnp.max(logits, axis=-1, keepdims=True)
    p = jnp.exp(logits - m)
    p = p / jnp.sum(p, axis=-1, keepdims=True)
    ii = lax.broadcasted_iota(jnp.int32, p.shape, 1)
    m1 = jnp.max(p, axis=-1, keepdims=True)
    i1 = jnp.min(jnp.where(p >= m1, ii, E), axis=-1, keepdims=True)
    p2 = jnp.where(ii == i1, -jnp.inf, p)
    m2 = jnp.max(p2, axis=-1, keepdims=True)
    i2 = jnp.min(jnp.where(p2 >= m2, ii, E), axis=-1, keepdims=True)
    s = m1 + m2
    w1 = m1 / s
    w2 = m2 / s

    ae = jnp.concatenate([i1, i2], axis=0)
    col = lax.broadcasted_iota(jnp.int32, (2 * BT, 128), 1)
    oh = (col == ae).astype(jnp.float32)
    r_io = lax.broadcasted_iota(jnp.int32, (2 * BT, 2 * BT), 0)
    c_io = lax.broadcasted_iota(jnp.int32, (2 * BT, 2 * BT), 1)
    tril = (c_io < r_io).astype(jnp.float32)
    local = jax.lax.dot_general(
        tril, oh, (((1,), (0,)), ((), ())), preferred_element_type=jnp.float32)
    base = cnt_ref[...]
    rk = local + base[0:1, :]
    rank = jnp.sum(jnp.where(col == ae, rk, 0.0), axis=1, keepdims=True)
    r1 = rank[:BT].astype(jnp.int32)
    r2 = rank[BT:].astype(jnp.int32)

    mcol = lax.broadcasted_iota(jnp.int32, (BT, 128), 1)
    mi_ref[...] = (jnp.where(mcol == 0, i1, 0) + jnp.where(mcol == 1, i2, 0)
                   + jnp.where(mcol == 2, r1, 0) + jnp.where(mcol == 3, r2, 0))
    mf_ref[...] = jnp.where(mcol == 0, w1, 0.0) + jnp.where(mcol == 1, w2, 0.0)

    newcnt = base + jnp.broadcast_to(
        jnp.sum(oh, axis=0, keepdims=True), base.shape)
    cnt_ref[...] = newcnt
    cnts_ref[...] = newcnt.astype(jnp.int32)


def _router(x, rw):
    return pl.pallas_call(
        _router_body,
        grid=(T // BT,),
        in_specs=[
            pl.BlockSpec((BT, D), lambda i: (i, 0)),
            pl.BlockSpec((E, D), lambda i: (0, 0)),
        ],
        out_specs=[
            pl.BlockSpec((BT, 128), lambda i: (i, 0)),
            pl.BlockSpec((BT, 128), lambda i: (i, 0)),
            pl.BlockSpec((8, 128), lambda i: (0, 0)),
        ],
        out_shape=[
            jax.ShapeDtypeStruct((T, 128), jnp.int32),
            jax.ShapeDtypeStruct((T, 128), jnp.float32),
            jax.ShapeDtypeStruct((8, 128), jnp.int32),
        ],
        scratch_shapes=[pltpu.VMEM((8, 128), jnp.float32)],
    )(x, rw)


def _dispatch(xb, dest, wsplat):
    mesh = plsc.VectorSubcoreMesh(core_axis_name="c", subcore_axis_name="s")

    @functools.partial(
        pl.kernel,
        mesh=mesh,
        out_type=[
            jax.ShapeDtypeStruct((NPAD, D), jnp.float32),
            jax.ShapeDtypeStruct((NPAD, 128), jnp.float32),
        ],
        scratch_types=[
            pltpu.VMEM((CH, D), jnp.float32),
            pltpu.VMEM((CH, 128), jnp.float32),
            pltpu.VMEM((CH,), jnp.int32),
            pltpu.SemaphoreType.DMA,
            pltpu.SemaphoreType.DMA,
        ],
    )
    def k(xb_hbm, dest_hbm, ws_hbm, xs_hbm, wp_hbm, rows_v, ws_v, idx_v,
          sem0, sem1):
        wid = lax.axis_index("s") * 2 + lax.axis_index("c")
        jbase = wid * (A // NW)
        for c in range(A // NW // CH):
            off = jbase + c * CH
            pltpu.sync_copy(dest_hbm.at[pl.ds(off, CH)], idx_v)
            pltpu.sync_copy(xb_hbm.at[pl.ds(off % T, CH)], rows_v)
            pltpu.sync_copy(ws_hbm.at[pl.ds(off, CH)], ws_v)
            cp0 = pltpu.async_copy(rows_v, xs_hbm.at[idx_v], sem0)
            cp1 = pltpu.async_copy(ws_v, wp_hbm.at[idx_v], sem1)
            cp0.wait()
            cp1.wait()

    return k(xb, dest, wsplat)


def _gemm_body(te_ref, x_ref, w13_ref, w2_ref, wp_ref, y_ref):
    x = x_ref[...]
    wg = w13_ref[0, :FF, :]
    wu = w13_ref[0, FF:, :]
    g = jax.lax.dot_general(
        x, wg, (((1,), (1,)), ((), ())), preferred_element_type=jnp.float32,
        precision=lax.Precision.DEFAULT)
    u = jax.lax.dot_general(
        x, wu, (((1,), (1,)), ((), ())), preferred_element_type=jnp.float32,
        precision=lax.Precision.DEFAULT)
    a = (g * jax.nn.sigmoid(g)) * u
    y = jax.lax.dot_general(
        a, w2_ref[0], (((1,), (1,)), ((), ())),
        preferred_element_type=jnp.float32,
        precision=lax.Precision.DEFAULT)
    wcol = lax.broadcasted_iota(jnp.int32, (BR, 128), 1)
    w = jnp.sum(jnp.where(wcol == 0, wp_ref[...], 0.0), axis=1, keepdims=True)
    y_ref[...] = y * w


def _gemm(te, xs, w13b, w2b, wpad):
    grid_spec = pltpu.PrefetchScalarGridSpec(
        num_scalar_prefetch=1,
        grid=(MT,),
        in_specs=[
            pl.BlockSpec((BR, D), lambda i, te: (i, 0)),
            pl.BlockSpec((1, 2 * FF, D), lambda i, te: (te[i], 0, 0)),
            pl.BlockSpec((1, D, FF), lambda i, te: (te[i], 0, 0)),
            pl.BlockSpec((BR, 128), lambda i, te: (i, 0)),
        ],
        out_specs=pl.BlockSpec((BR, D), lambda i, te: (i, 0)),
    )
    return pl.pallas_call(
        _gemm_body,
        grid_spec=grid_spec,
        out_shape=jax.ShapeDtypeStruct((NPAD, D), jnp.float32),
    )(te, xs, w13b, w2b, wpad)


def _combine(ys, dest):
    mesh = plsc.VectorSubcoreMesh(core_axis_name="c", subcore_axis_name="s")

    @functools.partial(
        pl.kernel,
        mesh=mesh,
        out_type=jax.ShapeDtypeStruct((T, D), jnp.float32),
        scratch_types=[
            pltpu.VMEM((CT,), jnp.int32),
            pltpu.VMEM((CT,), jnp.int32),
            pltpu.VMEM((CT, D), jnp.float32),
            pltpu.VMEM((CT, D), jnp.float32),
            pltpu.VMEM((CT, D), jnp.float32),
            pltpu.SemaphoreType.DMA,
            pltpu.SemaphoreType.DMA,
        ],
    )
    def k(ys_hbm, dest_hbm, out_hbm, idx0_v, idx1_v, r0_v, r1_v, o_v,
          sem0, sem1):
        wid = lax.axis_index("s") * 2 + lax.axis_index("c")
        tbase = wid * (T // NW)
        for c in range(T // NW // CT):
            tb = tbase + c * CT
            pltpu.sync_copy(dest_hbm.at[pl.ds(tb, CT)], idx0_v)
            pltpu.sync_copy(dest_hbm.at[pl.ds(T + tb, CT)], idx1_v)
            cp0 = pltpu.async_copy(ys_hbm.at[idx0_v], r0_v, sem0)
            cp1 = pltpu.async_copy(ys_hbm.at[idx1_v], r1_v, sem1)
            cp0.wait()
            cp1.wait()
            for r in range(CT):
                def body(cc, _):
                    sl = pl.ds(cc * 16, 16)
                    o_v[r, sl] = r0_v[r, sl] + r1_v[r, sl]
                    return 0
                lax.fori_loop(0, D // 16, body, 0)
            pltpu.sync_copy(o_v, out_hbm.at[pl.ds(tb, CT)])

    return k(ys, dest)


def kernel(hidden_states, router_weight, w13, w2):
    mi, mf, cnts = _router(hidden_states, router_weight)
    counts = cnts[0, :E]
    nt = (counts + BR - 1) // BR
    cum = jnp.cumsum(nt)
    po = (cum - nt) * BR
    ae = jnp.concatenate([mi[:, 0], mi[:, 1]])
    rank = jnp.concatenate([mi[:, 2], mi[:, 3]])
    dest = jnp.clip(po[ae] + rank, 0, NPAD - 1).astype(jnp.int32)
    te = jnp.clip(jnp.searchsorted(cum, jnp.arange(MT), side="right"),
                  0, E - 1).astype(jnp.int32)
    wsplat = jnp.broadcast_to(
        jnp.concatenate([mf[:, 0], mf[:, 1]])[:, None], (A, 128))

    xs, wpad = _dispatch(hidden_states, dest, wsplat)
    ys = _gemm(te, xs, w13, w2, wpad)
    return _combine(ys, dest)


# speedup vs baseline: 2.6753x; 1.0585x over previous
import functools

import jax
import jax.numpy as jnp
from jax import lax
from jax.experimental import pallas as pl
from jax.experimental.pallas import tpu as pltpu
from jax.experimental.pallas import tpu_sc as plsc

T, D, E, FF = 2048, 2048, 8, 1024
K = 2
A = T * K
BT = 256
BR = 256
MT = A // BR + E - 1
NPAD = MT * BR
NW = 32
CH = 16
CT = 8


def _router_body(x_ref, rw_ref, mi_ref, mf_ref, cnts_ref, cnt_ref):
    i = pl.program_id(0)

    @pl.when(i == 0)
    def _():
        cnt_ref[...] = jnp.zeros_like(cnt_ref)

    x = x_ref[...]
    rw = rw_ref[...]
    logits = jax.lax.dot_general(
        x, rw, (((1,), (1,)), ((), ())), preferred_element_type=jnp.float32)
    m = jnp.max(logits, axis=-1, keepdims=True)
    p = jnp.exp(logits - m)
    p = p / jnp.sum(p, axis=-1, keepdims=True)
    ii = lax.broadcasted_iota(jnp.int32, p.shape, 1)
    m1 = jnp.max(p, axis=-1, keepdims=True)
    i1 = jnp.min(jnp.where(p >= m1, ii, E), axis=-1, keepdims=True)
    p2 = jnp.where(ii == i1, -jnp.inf, p)
    m2 = jnp.max(p2, axis=-1, keepdims=True)
    i2 = jnp.min(jnp.where(p2 >= m2, ii, E), axis=-1, keepdims=True)
    s = m1 + m2
    w1 = m1 / s
    w2 = m2 / s

    ae = jnp.concatenate([i1, i2], axis=0)
    col = lax.broadcasted_iota(jnp.int32, (2 * BT, 128), 1)
    oh = (col == ae).astype(jnp.float32)
    r_io = lax.broadcasted_iota(jnp.int32, (2 * BT, 2 * BT), 0)
    c_io = lax.broadcasted_iota(jnp.int32, (2 * BT, 2 * BT), 1)
    tril = (c_io < r_io).astype(jnp.float32)
    local = jax.lax.dot_general(
        tril, oh, (((1,), (0,)), ((), ())), preferred_element_type=jnp.float32)
    base = cnt_ref[...]
    rk = local + base[0:1, :]
    rank = jnp.sum(jnp.where(col == ae, rk, 0.0), axis=1, keepdims=True)
    r1 = rank[:BT].astype(jnp.int32)
    r2 = rank[BT:].astype(jnp.int32)

    mcol = lax.broadcasted_iota(jnp.int32, (BT, 128), 1)
    mi_ref[...] = (jnp.where(mcol == 0, i1, 0) + jnp.where(mcol == 1, i2, 0)
                   + jnp.where(mcol == 2, r1, 0) + jnp.where(mcol == 3, r2, 0))
    mf_ref[...] = jnp.where(mcol == 0, w1, 0.0) + jnp.where(mcol == 1, w2, 0.0)

    newcnt = base + jnp.broadcast_to(
        jnp.sum(oh, axis=0, keepdims=True), base.shape)
    cnt_ref[...] = newcnt
    cnts_ref[...] = newcnt.astype(jnp.int32)


def _router(x, rw):
    return pl.pallas_call(
        _router_body,
        grid=(T // BT,),
        in_specs=[
            pl.BlockSpec((BT, D), lambda i: (i, 0)),
            pl.BlockSpec((E, D), lambda i: (0, 0)),
        ],
        out_specs=[
            pl.BlockSpec((BT, 128), lambda i: (i, 0)),
            pl.BlockSpec((BT, 128), lambda i: (i, 0)),
            pl.BlockSpec((8, 128), lambda i: (0, 0)),
        ],
        out_shape=[
            jax.ShapeDtypeStruct((T, 128), jnp.int32),
            jax.ShapeDtypeStruct((T, 128), jnp.float32),
            jax.ShapeDtypeStruct((8, 128), jnp.int32),
        ],
        scratch_shapes=[pltpu.VMEM((8, 128), jnp.float32)],
    )(x, rw)


def _dispatch(xb, dest, wsplat):
    mesh = plsc.VectorSubcoreMesh(core_axis_name="c", subcore_axis_name="s")

    @functools.partial(
        pl.kernel,
        mesh=mesh,
        out_type=[
            jax.ShapeDtypeStruct((NPAD, D), jnp.float32),
            jax.ShapeDtypeStruct((NPAD, 128), jnp.float32),
        ],
        scratch_types=[
            pltpu.VMEM((2, CH, D), jnp.float32),
            pltpu.VMEM((2, CH, 128), jnp.float32),
            pltpu.VMEM((2, CH), jnp.int32),
            pltpu.SemaphoreType.DMA,
            pltpu.SemaphoreType.DMA,
        ],
    )
    def k(xb_hbm, dest_hbm, ws_hbm, xs_hbm, wp_hbm, rows_v, ws_v, idx_v,
          sem_r, sem_w):
        wid = lax.axis_index("s") * 2 + lax.axis_index("c")
        jbase = wid * (A // NW)
        nch = A // NW // CH

        def start_reads(c, b):
            off = jbase + c * CH
            return (pltpu.async_copy(dest_hbm.at[pl.ds(off, CH)],
                                     idx_v.at[b], sem_r),
                    pltpu.async_copy(xb_hbm.at[pl.ds(off % T, CH)],
                                     rows_v.at[b], sem_r),
                    pltpu.async_copy(ws_hbm.at[pl.ds(off, CH)],
                                     ws_v.at[b], sem_r))

        reads = start_reads(0, 0)
        scats = [None, None]
        for c in range(nch):
            b = c % 2
            for r in reads:
                r.wait()
            if c + 1 < nch:
                reads = start_reads(c + 1, 1 - b)
            if scats[b] is not None:
                for sc in scats[b]:
                    sc.wait()
            scats[b] = (
                pltpu.async_copy(rows_v.at[b], xs_hbm.at[idx_v.at[b]], sem_w),
                pltpu.async_copy(ws_v.at[b], wp_hbm.at[idx_v.at[b]], sem_w))
        for b in range(2):
            if scats[b] is not None:
                for sc in scats[b]:
                    sc.wait()

    return k(xb, dest, wsplat)


def _gemm_body(te_ref, x_ref, w13_ref, w2_ref, wp_ref, y_ref):
    x = x_ref[...]
    wg = w13_ref[0, :FF, :]
    wu = w13_ref[0, FF:, :]
    g = jax.lax.dot_general(
        x, wg, (((1,), (1,)), ((), ())), preferred_element_type=jnp.float32,
        precision=lax.Precision.DEFAULT)
    u = jax.lax.dot_general(
        x, wu, (((1,), (1,)), ((), ())), preferred_element_type=jnp.float32,
        precision=lax.Precision.DEFAULT)
    a = (g * jax.nn.sigmoid(g)) * u
    y = jax.lax.dot_general(
        a, w2_ref[0], (((1,), (1,)), ((), ())),
        preferred_element_type=jnp.float32,
        precision=lax.Precision.DEFAULT)
    wcol = lax.broadcasted_iota(jnp.int32, (BR, 128), 1)
    w = jnp.sum(jnp.where(wcol == 0, wp_ref[...], 0.0), axis=1, keepdims=True)
    y_ref[...] = y * w


def _gemm(te, xs, w13b, w2b, wpad):
    grid_spec = pltpu.PrefetchScalarGridSpec(
        num_scalar_prefetch=1,
        grid=(MT,),
        in_specs=[
            pl.BlockSpec((BR, D), lambda i, te: (i, 0)),
            pl.BlockSpec((1, 2 * FF, D), lambda i, te: (te[i], 0, 0)),
            pl.BlockSpec((1, D, FF), lambda i, te: (te[i], 0, 0)),
            pl.BlockSpec((BR, 128), lambda i, te: (i, 0)),
        ],
        out_specs=pl.BlockSpec((BR, D), lambda i, te: (i, 0)),
    )
    return pl.pallas_call(
        _gemm_body,
        grid_spec=grid_spec,
        out_shape=jax.ShapeDtypeStruct((NPAD, D), jnp.float32),
    )(te, xs, w13b, w2b, wpad)


def _combine(ys, dest):
    mesh = plsc.VectorSubcoreMesh(core_axis_name="c", subcore_axis_name="s")

    @functools.partial(
        pl.kernel,
        mesh=mesh,
        out_type=jax.ShapeDtypeStruct((T, D), jnp.float32),
        scratch_types=[
            pltpu.VMEM((2, CT), jnp.int32),
            pltpu.VMEM((2, CT), jnp.int32),
            pltpu.VMEM((2, CT, D), jnp.float32),
            pltpu.VMEM((2, CT, D), jnp.float32),
            pltpu.VMEM((2, CT, D), jnp.float32),
            pltpu.SemaphoreType.DMA,
            pltpu.SemaphoreType.DMA,
        ],
    )
    def k(ys_hbm, dest_hbm, out_hbm, idx0_v, idx1_v, r0_v, r1_v, o_v,
          sem_r, sem_w):
        wid = lax.axis_index("s") * 2 + lax.axis_index("c")
        tbase = wid * (T // NW)
        nch = T // NW // CT

        def start_gathers(c, b):
            tb = tbase + c * CT
            pltpu.sync_copy(dest_hbm.at[pl.ds(tb, CT)], idx0_v.at[b])
            pltpu.sync_copy(dest_hbm.at[pl.ds(T + tb, CT)], idx1_v.at[b])
            return (pltpu.async_copy(ys_hbm.at[idx0_v.at[b]], r0_v.at[b],
                                     sem_r),
                    pltpu.async_copy(ys_hbm.at[idx1_v.at[b]], r1_v.at[b],
                                     sem_r))

        gath = start_gathers(0, 0)
        writes = [None, None]
        for c in range(nch):
            b = c % 2
            for g in gath:
                g.wait()
            if c + 1 < nch:
                gath = start_gathers(c + 1, 1 - b)
            if writes[b] is not None:
                writes[b].wait()
            for r in range(CT):
                def body(cc, _):
                    sl = pl.ds(cc * 16, 16)
                    o_v[b, r, sl] = r0_v[b, r, sl] + r1_v[b, r, sl]
                    return 0
                lax.fori_loop(0, D // 16, body, 0)
            writes[b] = pltpu.async_copy(
                o_v.at[b], out_hbm.at[pl.ds(tbase + c * CT, CT)], sem_w)
        for b in range(2):
            if writes[b] is not None:
                writes[b].wait()

    return k(ys, dest)


def kernel(hidden_states, router_weight, w13, w2):
    mi, mf, cnts = _router(hidden_states, router_weight)
    counts = cnts[0, :E]
    nt = (counts + BR - 1) // BR
    cum = jnp.cumsum(nt)
    po = (cum - nt) * BR
    ae = jnp.concatenate([mi[:, 0], mi[:, 1]])
    rank = jnp.concatenate([mi[:, 2], mi[:, 3]])
    dest = jnp.clip(po[ae] + rank, 0, NPAD - 1).astype(jnp.int32)
    te = jnp.clip(jnp.searchsorted(cum, jnp.arange(MT), side="right"),
                  0, E - 1).astype(jnp.int32)
    wsplat = jnp.broadcast_to(
        jnp.concatenate([mf[:, 0], mf[:, 1]])[:, None], (A, 128))

    xs, wpad = _dispatch(hidden_states, dest, wsplat)
    ys = _gemm(te, xs, w13, w2, wpad)
    return _combine(ys, dest)
